# Initial kernel scaffold; baseline (speedup 1.0000x reference)
import functools

import jax
import jax.numpy as jnp
from jax import lax
from jax.experimental import pallas as pl
from jax.experimental.pallas import tpu as pltpu
from jax.experimental.pallas import tpu_sc as plsc

VOCAB = 100000
D = 64
B = 4096 * 50
NC = 2
NS = 16
NW = NC * NS
BPW = B // NW
CH = 128
NCH = BPW // CH

_mesh = plsc.VectorSubcoreMesh(core_axis_name="c", subcore_axis_name="s")


@functools.partial(
    pl.kernel,
    mesh=_mesh,
    out_type=jax.ShapeDtypeStruct((NW, NCH, CH, D), jnp.float32),
    scratch_types=[
        pltpu.VMEM((NCH, CH), jnp.int32),
        pltpu.VMEM((CH, D), jnp.float32),
        pltpu.SemaphoreType.DMA,
    ],
)
def _gather(table_hbm, idx_hbm, out_hbm, idx_v, rows_v, sem):
    wid = lax.axis_index("s") * NC + lax.axis_index("c")
    pltpu.sync_copy(idx_hbm.at[wid], idx_v)

    def body(j, carry):
        pltpu.async_copy(table_hbm.at[idx_v.at[j]], rows_v, sem).wait()
        pltpu.sync_copy(rows_v, out_hbm.at[wid, j])
        return carry

    lax.fori_loop(0, NCH, body, 0)


def kernel(input_ids, embedding_matrix):
    ids = input_ids.reshape(NW, NCH, CH).astype(jnp.int32)
    out = _gather(embedding_matrix, ids)
    return out.reshape(input_ids.shape + (D,))


# speedup vs baseline: 4.0854x; 4.0854x over previous
import functools

import jax
import jax.numpy as jnp
from jax import lax
from jax.experimental import pallas as pl
from jax.experimental.pallas import tpu as pltpu
from jax.experimental.pallas import tpu_sc as plsc

VOCAB = 100000
D = 64
B = 4096 * 50
NC = 2
NS = 16
NW = NC * NS
BPW = B // NW
CH = 128
NCH = BPW // CH

_mesh = plsc.VectorSubcoreMesh(core_axis_name="c", subcore_axis_name="s")


@functools.partial(
    pl.kernel,
    mesh=_mesh,
    out_type=jax.ShapeDtypeStruct((NW, NCH, CH, D), jnp.float32),
    scratch_types=[
        pltpu.VMEM((NCH, CH), jnp.int32),
        pltpu.VMEM((CH, D), jnp.float32),
        pltpu.SemaphoreType.DMA,
    ],
    compiler_params=pltpu.CompilerParams(use_tc_tiling_on_sc=False),
)
def _gather(table_hbm, idx_hbm, out_hbm, idx_v, rows_v, sem):
    wid = lax.axis_index("s") * NC + lax.axis_index("c")
    pltpu.sync_copy(idx_hbm.at[wid], idx_v)

    def body(j, carry):
        pltpu.async_copy(table_hbm.at[idx_v.at[j]], rows_v, sem).wait()
        pltpu.sync_copy(rows_v, out_hbm.at[wid, j])
        return carry

    lax.fori_loop(0, NCH, body, 0)


def kernel(input_ids, embedding_matrix):
    ids = input_ids.reshape(NW, NCH, CH).astype(jnp.int32)
    out = _gather(embedding_matrix, ids)
    return out.reshape(input_ids.shape + (D,))


# speedup vs baseline: 4.6711x; 1.1434x over previous
import functools

import jax
import jax.numpy as jnp
from jax import lax
from jax.experimental import pallas as pl
from jax.experimental.pallas import tpu as pltpu
from jax.experimental.pallas import tpu_sc as plsc

VOCAB = 100000
D = 64
B = 4096 * 50
NC = 2
NS = 16
NW = NC * NS
BPW = B // NW
CH = 128
NCH = BPW // CH

_mesh = plsc.VectorSubcoreMesh(core_axis_name="c", subcore_axis_name="s")


NBUF = 5


@functools.partial(
    pl.kernel,
    mesh=_mesh,
    out_type=jax.ShapeDtypeStruct((NW, NCH, CH, D), jnp.float32),
    scratch_types=[
        pltpu.VMEM((NCH, CH), jnp.int32),
        pltpu.VMEM((NBUF, CH, D), jnp.float32),
        [pltpu.SemaphoreType.DMA] * NBUF,
        [pltpu.SemaphoreType.DMA] * NBUF,
    ],
    compiler_params=pltpu.CompilerParams(use_tc_tiling_on_sc=False),
)
def _gather(table_hbm, idx_hbm, out_hbm, idx_v, rows_v, gsems, ssems):
    wid = lax.axis_index("s") * NC + lax.axis_index("c")
    pltpu.sync_copy(idx_hbm.at[wid], idx_v)

    for b in range(NBUF):
        pltpu.async_copy(table_hbm.at[idx_v.at[b]], rows_v.at[b], gsems[b])

    def outer(jo, carry):
        for b in range(NBUF):
            j = jo + b
            pltpu.make_async_copy(table_hbm.at[idx_v.at[b]], rows_v.at[b],
                                  gsems[b]).wait()
            pltpu.async_copy(rows_v.at[b], out_hbm.at[wid, j], ssems[b])
            pltpu.make_async_copy(rows_v.at[b], out_hbm.at[wid, j],
                                  ssems[b]).wait()
            pltpu.async_copy(table_hbm.at[idx_v.at[j + NBUF]], rows_v.at[b],
                             gsems[b])
        return carry

    lax.fori_loop(0, (NCH - NBUF) // NBUF, lambda i, c: outer(i * NBUF, c), 0)

    for b in range(NBUF):
        j = NCH - NBUF + b
        pltpu.make_async_copy(table_hbm.at[idx_v.at[b]], rows_v.at[b],
                              gsems[b]).wait()
        pltpu.sync_copy(rows_v.at[b], out_hbm.at[wid, j])


def kernel(input_ids, embedding_matrix):
    ids = input_ids.reshape(NW, NCH, CH).astype(jnp.int32)
    out = _gather(embedding_matrix, ids)
    return out.reshape(input_ids.shape + (D,))


# speedup vs baseline: 4.6788x; 1.0017x over previous
import functools

import jax
import jax.numpy as jnp
from jax import lax
from jax.experimental import pallas as pl
from jax.experimental.pallas import tpu as pltpu
from jax.experimental.pallas import tpu_sc as plsc

VOCAB = 100000
D = 64
B = 4096 * 50
NC = 2
NS = 16
NW = NC * NS
BPW = B // NW
CH = 128
NCH = BPW // CH

_mesh = plsc.VectorSubcoreMesh(core_axis_name="c", subcore_axis_name="s")


NBUF = 10
HALF = NBUF // 2


@functools.partial(
    pl.kernel,
    mesh=_mesh,
    out_type=jax.ShapeDtypeStruct((NW, NCH, CH, D), jnp.float32),
    scratch_types=[
        pltpu.VMEM((NCH, CH), jnp.int32),
        pltpu.VMEM((NBUF, CH, D), jnp.float32),
        [pltpu.SemaphoreType.DMA] * NBUF,
        [pltpu.SemaphoreType.DMA] * NBUF,
    ],
    compiler_params=pltpu.CompilerParams(use_tc_tiling_on_sc=False),
)
def _gather(table_hbm, idx_hbm, out_hbm, idx_v, rows_v, gsems, ssems):
    wid = lax.axis_index("s") * NC + lax.axis_index("c")
    pltpu.sync_copy(idx_hbm.at[wid], idx_v)

    def ig(j, b):
        pltpu.async_copy(table_hbm.at[idx_v.at[j]], rows_v.at[b], gsems[b])

    def wg(j, b):
        pltpu.make_async_copy(table_hbm.at[idx_v.at[j]], rows_v.at[b],
                              gsems[b]).wait()

    def iscat(j, b):
        pltpu.async_copy(rows_v.at[b], out_hbm.at[wid, j], ssems[b])

    def wscat(j, b):
        pltpu.make_async_copy(rows_v.at[b], out_hbm.at[wid, j],
                              ssems[b]).wait()

    for b in range(NBUF):
        ig(b, b)
    for b in range(NBUF):
        wg(b, b)
        iscat(b, b)
        if b >= HALF:
            b2 = (b + HALF) % NBUF
            wscat(b - HALF, b2)
            ig(b + HALF, b2)

    def mid(p, carry):
        for b in range(NBUF):
            j = p * NBUF + b
            wg(j, b)
            iscat(j, b)
            b2 = (b + HALF) % NBUF
            wscat(j - HALF, b2)
            ig(j + HALF, b2)
        return carry

    lax.fori_loop(1, NCH // NBUF - 1, mid, 0)

    for b in range(NBUF):
        j = NCH - NBUF + b
        wg(j, b)
        iscat(j, b)
        if b < HALF:
            b2 = (b + HALF) % NBUF
            wscat(j - HALF, b2)
            ig(j + HALF, b2)
    for b in range(NBUF):
        wscat(NCH - NBUF + b, b)


def kernel(input_ids, embedding_matrix):
    ids = input_ids.reshape(NW, NCH, CH).astype(jnp.int32)
    out = _gather(embedding_matrix, ids)
    return out.reshape(input_ids.shape + (D,))
